# baseline (speedup 1.0000x reference)
import jax
import jax.numpy as jnp
from jax.experimental import pallas as pl

N = 10000
E = 160000
B = 4
D = 64
INPUT_DIM = 64
PRE_K = 4
PRE_V = 2
KDIFF = 2


def _spmm(sr, sc, sv, X):
    return jax.ops.segment_sum(sv[:, None] * X[sc], sr, num_segments=N)


def _tail_body(conv_ref, w_ref, b_ref, att_ref, out_ref):
    c = conv_ref[...]
    c = jnp.where(c >= 0, c, 0.01 * c)
    out_ref[...] = jnp.dot(c, w_ref[...], preferred_element_type=jnp.float32) + b_ref[...] + att_ref[...]


def kernel(inputs, hx_k, adj_vals, rows, cols, gconv_w, gconv_b, W, b, R, att_w, att_b):
    deg = jax.ops.segment_sum(adj_vals, rows, num_segments=N)
    dinv = jnp.where(deg > 0, 1.0 / jnp.where(deg > 0, deg, 1.0), 0.0)
    degt = jax.ops.segment_sum(adj_vals, cols, num_segments=N)
    dtinv = jnp.where(degt > 0, 1.0 / jnp.where(degt > 0, degt, 1.0), 0.0)
    v1 = adj_vals * dinv[rows]
    v2 = adj_vals * dtinv[cols]

    preH = jnp.concatenate([hx_k[:, PRE_K - 1], hx_k[:, PRE_K - 2]], axis=-1)
    x = jnp.concatenate([inputs.reshape(B, N, INPUT_DIM), preH], axis=2)
    input_size = x.shape[2]
    x0 = jnp.transpose(x, (1, 2, 0)).reshape(N, input_size * B)

    y1 = _spmm(cols, rows, v1, x0)
    y2 = 2.0 * _spmm(cols, rows, v1, y1) - x0
    y3 = _spmm(rows, cols, v2, y1)
    y4 = 2.0 * _spmm(rows, cols, v2, y3) - y1

    xst = jnp.stack([x0, y1, y2, y3, y4], axis=0)
    num_matrices = 5
    xst = xst.reshape(num_matrices, N, input_size, B)
    xst = jnp.transpose(xst, (3, 1, 2, 0)).reshape(B * N, input_size * num_matrices)
    conv = (jnp.matmul(xst, gconv_w) + gconv_b).reshape(B, N, D)

    new_states = hx_k + R[None]
    logits = jnp.matmul(new_states.reshape(B, PRE_K, N * D), att_w) + att_b
    weight = jax.nn.softmax(logits, axis=1)
    att = (new_states.reshape(B, PRE_K, N * D) * weight).sum(axis=1).reshape(B, N, D)

    BN = 400
    out = pl.pallas_call(
        _tail_body,
        grid=(B, N // BN),
        in_specs=[
            pl.BlockSpec((1, BN, D), lambda bb, nb: (bb, nb, 0)),
            pl.BlockSpec((D, D), lambda bb, nb: (0, 0)),
            pl.BlockSpec((BN, D), lambda bb, nb: (nb, 0)),
            pl.BlockSpec((1, BN, D), lambda bb, nb: (bb, nb, 0)),
        ],
        out_specs=pl.BlockSpec((1, BN, D), lambda bb, nb: (bb, nb, 0)),
        out_shape=jax.ShapeDtypeStruct((B, N, D), jnp.float32),
    )(conv, W, b, att)

    hx_new = jnp.concatenate([hx_k[:, 1:PRE_K], out[:, None]], axis=1)
    return out.reshape(B, N * D), hx_new


# speedup vs baseline: 3.6812x; 3.6812x over previous
import functools
import jax
import jax.numpy as jnp
from jax import lax
from jax.experimental import pallas as pl
from jax.experimental.pallas import tpu as pltpu
from jax.experimental.pallas import tpu_sc as plsc

N = 10000
E = 160000
B = 4
D = 64
INPUT_DIM = 64
PRE_K = 4
PRE_V = 2

NC, NS, L = 2, 16, 16
IS = 192
FC = 128
NCHUNK = 6
CPS = NCHUNK // NC
EP = E // NS
EPP = 10080
KB = 32
NG = EPP // KB
NBUF = 5
NOG = NG // NBUF
WB = 40
RB = 640
RBLK = 80
FCV = FC // L


def _scale_rows(buf, vv, b):
    @plsc.parallel_loop(0, KB // L)
    def _(j):
        vvv = vv[b, pl.ds(j * L, L)]
        for l in range(L):
            v = jnp.full((L,), vvv[l], jnp.float32)
            r = j * L + l
            for k in range(FCV):
                buf[r, pl.ds(k * L, L)] = buf[r, pl.ds(k * L, L)] * v


def _sc_body(x0_hbm, rows_hbm, cols_hbm, vals_hbm,
             y1_hbm, y2_hbm, y3_hbm, y4_hbm,
             acc, deg_s, degt_s,
             gi_all, si_all, adj_all, dval, gidx2, sidx2, didx2,
             gbuf, zb, rb,
             esem, dsems, gsems, ssems, wsems):
    cid = lax.axis_index("c")
    sid = lax.axis_index("s")
    ebase = sid * EPP
    nblk2 = jnp.where(sid == NS - 1, 5, 8)
    nrb = jnp.where(sid == NS - 1, 5, 8)

    @plsc.parallel_loop(0, WB)
    def _(r):
        for k in range(FCV):
            zb[r, pl.ds(k * L, L)] = jnp.zeros((L,), jnp.float32)

    for i in range(RBLK // L):
        rb[pl.ds(i * L, L)] = jnp.zeros((L,), jnp.float32)

    def dz_body(w, _):
        r0 = sid * RB + w * RBLK
        pltpu.sync_copy(rb, deg_s.at[pl.ds(r0, RBLK)])
        pltpu.sync_copy(rb, degt_s.at[pl.ds(r0, RBLK)])
        return 0
    lax.fori_loop(0, nrb, dz_body, 0)
    plsc.subcore_barrier()

    def deg_og(og, _):
        e0 = ebase + og * (NBUF * KB)
        el = [pltpu.async_copy(rows_hbm.at[pl.ds(e0, NBUF * KB)], gi_all, esem),
              pltpu.async_copy(cols_hbm.at[pl.ds(e0, NBUF * KB)], si_all, esem),
              pltpu.async_copy(vals_hbm.at[pl.ds(e0, NBUF * KB)], adj_all, esem)]
        for d in el:
            d.wait()
        sds = []
        for b in range(NBUF):
            for k in range(KB // L):
                sl2 = pl.ds(k * L, L)
                sl1 = pl.ds(b * KB + k * L, L)
                gidx2[b, sl2] = gi_all[sl1]
                sidx2[b, sl2] = si_all[sl1]
                dval[b, sl2] = adj_all[sl1]
            sds.append(pltpu.async_copy(
                dval.at[b], deg_s.at[gidx2.at[b]], gsems[b], add=True))
            sds.append(pltpu.async_copy(
                dval.at[b], degt_s.at[sidx2.at[b]], ssems[b], add=True))
        for d in sds:
            d.wait()
        return 0
    lax.fori_loop(0, NOG, deg_og, 0)
    plsc.subcore_barrier()

    def recip(seg):
        def rbod(w, _):
            r0 = sid * RB + w * RBLK
            pltpu.sync_copy(seg.at[pl.ds(r0, RBLK)], rb)
            for i in range(RBLK // L):
                sl = pl.ds(i * L, L)
                v = rb[sl]
                rb[sl] = jnp.where(v > 0.0, 1.0 / jnp.where(v > 0.0, v, 1.0), 0.0)
            pltpu.sync_copy(rb, seg.at[pl.ds(r0, RBLK)])
            return 0
        lax.fori_loop(0, nrb, rbod, 0)

    recip(deg_s)
    recip(degt_s)
    plsc.subcore_barrier()

    def run_stage(gi_hbm, si_hbm, dinv_s, src_hbm, dst_hbm, cn):
        def zbod(w, _):
            pltpu.sync_copy(zb, acc.at[pl.ds(sid * RB + w * WB, WB)])
            return 0
        lax.fori_loop(0, 2 * nblk2, zbod, 0)
        plsc.subcore_barrier()

        def og_body(og, _):
            e0 = ebase + og * (NBUF * KB)
            el = [pltpu.async_copy(gi_hbm.at[pl.ds(e0, NBUF * KB)], gi_all, esem),
                  pltpu.async_copy(si_hbm.at[pl.ds(e0, NBUF * KB)], si_all, esem),
                  pltpu.async_copy(vals_hbm.at[pl.ds(e0, NBUF * KB)], adj_all, esem)]
            for d in el:
                d.wait()
            dds, gds = [], []
            for b in range(NBUF):
                for k in range(KB // L):
                    sl2 = pl.ds(k * L, L)
                    sl1 = pl.ds(b * KB + k * L, L)
                    giv = gi_all[sl1]
                    gidx2[b, sl2] = giv + cn
                    didx2[b, sl2] = giv
                    sidx2[b, sl2] = si_all[sl1]
                dds.append(pltpu.async_copy(
                    dinv_s.at[didx2.at[b]], dval.at[b], dsems[b]))
                gds.append(pltpu.async_copy(
                    src_hbm.at[gidx2.at[b]], gbuf.at[b], gsems[b]))
            sds = []
            for b in range(NBUF):
                dds[b].wait()
                gds[b].wait()
                for k in range(KB // L):
                    sl2 = pl.ds(k * L, L)
                    dval[b, sl2] = dval[b, sl2] * adj_all[pl.ds(b * KB + k * L, L)]
                _scale_rows(gbuf.at[b], dval, b)
                sds.append(pltpu.async_copy(
                    gbuf.at[b], acc.at[sidx2.at[b]], ssems[b], add=True))
            for d in sds:
                d.wait()
            return 0
        lax.fori_loop(0, NOG, og_body, 0)
        plsc.subcore_barrier()

        def wbod(w, _):
            ds0 = []
            for j in range(2):
                r0 = sid * RB + (w * 2 + j) * WB
                ds0.append(pltpu.async_copy(
                    acc.at[pl.ds(r0, WB)], dst_hbm.at[pl.ds(cn + r0, WB)], wsems[j]))
            for d in ds0:
                d.wait()
            return 0
        lax.fori_loop(0, nblk2, wbod, 0)
        plsc.subcore_barrier()

    def chunk_body(q, _):
        cn = (cid * CPS + q) * N
        run_stage(rows_hbm, cols_hbm, deg_s, x0_hbm, y1_hbm, cn)
        run_stage(rows_hbm, cols_hbm, deg_s, y1_hbm, y2_hbm, cn)
        run_stage(cols_hbm, rows_hbm, degt_s, y1_hbm, y3_hbm, cn)
        run_stage(cols_hbm, rows_hbm, degt_s, y3_hbm, y4_hbm, cn)
        return 0
    lax.fori_loop(0, CPS, chunk_body, 0)


_sc_diffusion = functools.partial(
    pl.kernel,
    out_type=[jax.ShapeDtypeStruct((NCHUNK * N, FC), jnp.float32)] * 4,
    mesh=plsc.VectorSubcoreMesh(core_axis_name="c", subcore_axis_name="s"),
    compiler_params=pltpu.CompilerParams(needs_layout_passes=False),
    scratch_types=[
        pltpu.VMEM_SHARED((N, FC), jnp.float32),
        pltpu.VMEM_SHARED((N,), jnp.float32),
        pltpu.VMEM_SHARED((N,), jnp.float32),
        pltpu.VMEM((NBUF * KB,), jnp.int32),
        pltpu.VMEM((NBUF * KB,), jnp.int32),
        pltpu.VMEM((NBUF * KB,), jnp.float32),
        pltpu.VMEM((NBUF, KB), jnp.float32),
        pltpu.VMEM((NBUF, KB), jnp.int32),
        pltpu.VMEM((NBUF, KB), jnp.int32),
        pltpu.VMEM((NBUF, KB), jnp.int32),
        pltpu.VMEM((NBUF, KB, FC), jnp.float32),
        pltpu.VMEM((WB, FC), jnp.float32),
        pltpu.VMEM((RBLK,), jnp.float32),
        pltpu.SemaphoreType.DMA,
        [pltpu.SemaphoreType.DMA] * NBUF,
        [pltpu.SemaphoreType.DMA] * NBUF,
        [pltpu.SemaphoreType.DMA] * NBUF,
        [pltpu.SemaphoreType.DMA] * 2,
    ],
)(_sc_body)


def _tail_body(conv_ref, w_ref, b_ref, att_ref, out_ref):
    c = conv_ref[...]
    c = jnp.where(c >= 0, c, 0.01 * c)
    out_ref[...] = jnp.dot(c, w_ref[...], preferred_element_type=jnp.float32) + b_ref[...] + att_ref[...]


def kernel(inputs, hx_k, adj_vals, rows, cols, gconv_w, gconv_b, W, b, R, att_w, att_b):
    preH = jnp.concatenate([hx_k[:, PRE_K - 1], hx_k[:, PRE_K - 2]], axis=-1)
    x = jnp.concatenate([inputs.reshape(B, N, INPUT_DIM), preH], axis=2)
    x0c = (x.transpose(1, 0, 2).reshape(N, NCHUNK, FC)
           .transpose(1, 0, 2).reshape(NCHUNK * N, FC))
    pad = ((0, 0), (0, EPP - EP))
    rows1 = jnp.pad(rows.astype(jnp.int32).reshape(NS, EP), pad).reshape(-1)
    cols1 = jnp.pad(cols.astype(jnp.int32).reshape(NS, EP), pad).reshape(-1)
    vals1 = jnp.pad(adj_vals.reshape(NS, EP), pad).reshape(-1)

    y1c, s2c, y3c, s4c = _sc_diffusion(x0c, rows1, cols1, vals1)

    def unchunk(a):
        return (a.reshape(NCHUNK, N, FC).transpose(1, 0, 2)
                .reshape(N, B, IS).transpose(1, 0, 2))

    Wm = gconv_w.reshape(IS, 5, D)
    Weff = jnp.stack([Wm[:, 0] - Wm[:, 2], Wm[:, 1] - Wm[:, 4],
                      2.0 * Wm[:, 2], Wm[:, 3], 2.0 * Wm[:, 4]], axis=0)
    xs = jnp.stack([unchunk(x0c), unchunk(y1c), unchunk(s2c),
                    unchunk(y3c), unchunk(s4c)], axis=0)
    conv = jnp.einsum('mbni,mid->bnd', xs, Weff) + gconv_b

    new_states = hx_k + R[None]
    logits = jnp.matmul(new_states.reshape(B, PRE_K, N * D), att_w) + att_b
    weight = jax.nn.softmax(logits, axis=1)
    att = (new_states.reshape(B, PRE_K, N * D) * weight).sum(axis=1).reshape(B, N, D)

    BN = 400
    out = pl.pallas_call(
        _tail_body,
        grid=(B, N // BN),
        in_specs=[
            pl.BlockSpec((1, BN, D), lambda bb, nb: (bb, nb, 0)),
            pl.BlockSpec((D, D), lambda bb, nb: (0, 0)),
            pl.BlockSpec((BN, D), lambda bb, nb: (nb, 0)),
            pl.BlockSpec((1, BN, D), lambda bb, nb: (bb, nb, 0)),
        ],
        out_specs=pl.BlockSpec((1, BN, D), lambda bb, nb: (bb, nb, 0)),
        out_shape=jax.ShapeDtypeStruct((B, N, D), jnp.float32),
    )(conv, W, b, att)

    hx_new = jnp.concatenate([hx_k[:, 1:PRE_K], out[:, None]], axis=1)
    return out.reshape(B, N * D), hx_new
